# Initial kernel scaffold; baseline (speedup 1.0000x reference)
import jax
import jax.numpy as jnp
from jax.experimental import pallas as pl


def kernel(x, w1, b1, w2, b2, dw_w, dw_b):
    raise NotImplementedError("write your pallas kernel here")


# speedup vs baseline: 1.6181x; 1.6181x over previous
import math

import jax
import jax.numpy as jnp
from jax.experimental import pallas as pl
from jax.experimental.pallas import tpu as pltpu


def _ffn_kernel(x_ref, w1_ref, b1_ref, dww_ref, dwb_ref, w2_ref, b2_ref,
                o_ref, *, H, W):
    N = H * W

    h = jnp.dot(x_ref[0], w1_ref[...], preferred_element_type=jnp.float32)
    h = h + b1_ref[...].astype(jnp.float32)
    hid = h.shape[-1]
    hw = h.reshape(H, W, hid)

    zrow = jnp.zeros((1, W, hid), jnp.float32)
    xp = jnp.concatenate([zrow, hw, zrow], axis=0)
    zcol = jnp.zeros((H + 2, 1, hid), jnp.float32)
    xp = jnp.concatenate([zcol, xp, zcol], axis=1)

    wk = dww_ref[...].astype(jnp.float32)
    conv = dwb_ref[...].astype(jnp.float32)
    for dh in range(3):
        for dw in range(3):
            conv = conv + xp[dh:dh + H, dw:dw + W, :] * wk[dh:dh + 1, dw:dw + 1, :]

    inv_sqrt2 = jnp.float32(0.7071067811865476)
    g = 0.5 * conv * (1.0 + jax.lax.erf(conv * inv_sqrt2))

    g2 = g.reshape(N, hid).astype(w2_ref.dtype)
    out = jnp.dot(g2, w2_ref[...], preferred_element_type=jnp.float32)
    o_ref[0] = (out + b2_ref[...].astype(jnp.float32)).astype(o_ref.dtype)


def _fused_ffn(x, w1, b1, dw_w, dw_b, w2, b2, *, H, W, interpret=False):
    B, N, C = x.shape
    hid = w1.shape[1]
    assert N == H * W

    def body(*refs):
        _ffn_kernel(*refs, H=H, W=W)

    return pl.pallas_call(
        body,
        out_shape=jax.ShapeDtypeStruct((B, N, C), jnp.float32),
        grid_spec=pltpu.PrefetchScalarGridSpec(
            num_scalar_prefetch=0,
            grid=(B,),
            in_specs=[
                pl.BlockSpec((1, N, C), lambda b: (b, 0, 0)),
                pl.BlockSpec((C, hid), lambda b: (0, 0)),
                pl.BlockSpec((1, hid), lambda b: (0, 0)),
                pl.BlockSpec((3, 3, hid), lambda b: (0, 0, 0)),
                pl.BlockSpec((1, 1, hid), lambda b: (0, 0, 0)),
                pl.BlockSpec((hid, C), lambda b: (0, 0)),
                pl.BlockSpec((1, C), lambda b: (0, 0)),
            ],
            out_specs=pl.BlockSpec((1, N, C), lambda b: (b, 0, 0)),
        ),
        compiler_params=pltpu.CompilerParams(
            dimension_semantics=("parallel",),
            vmem_limit_bytes=64 * 1024 * 1024,
        ),
        cost_estimate=pl.CostEstimate(
            flops=2 * B * N * C * hid * 2 + 18 * B * N * hid,
            transcendentals=B * N * hid,
            bytes_accessed=(B * N * C * 2 + B * N * C * 4
                            + (C * hid + hid * C) * 2),
        ),
        interpret=interpret,
    )(x, w1, b1.reshape(1, hid), dw_w, dw_b.reshape(1, 1, hid),
      w2, b2.reshape(1, C))


def kernel(x, w1, b1, w2, b2, dw_w, dw_b):
    B, N, C = x.shape
    H = W = math.isqrt(N)
    return _fused_ffn(
        x.astype(jnp.bfloat16),
        w1.astype(jnp.bfloat16), b1,
        dw_w, dw_b,
        w2.astype(jnp.bfloat16), b2,
        H=H, W=W,
    )


# speedup vs baseline: 2.0158x; 1.2458x over previous
import math

import jax
import jax.numpy as jnp
from jax.experimental import pallas as pl
from jax.experimental.pallas import tpu as pltpu


def _ffn_kernel(x_ref, w3a_ref, w3b_ref, w3c_ref, bias_ref, w2_ref, b2_ref,
                o_ref, *, H, W):
    N = H * W
    C = x_ref.shape[-1]

    x3 = x_ref[0].reshape(H, W, C)
    zc = jnp.zeros((H, 1, C), x3.dtype)
    xm = jnp.concatenate([zc, x3[:, :W - 1]], axis=1)
    xp = jnp.concatenate([x3[:, 1:], zc], axis=1)
    x9 = jnp.concatenate([xm, x3, xp], axis=2).reshape(N, 3 * C)

    conv = jnp.dot(x9, w3b_ref[...], preferred_element_type=jnp.float32)
    hid = conv.shape[-1]
    t_top = jnp.dot(x9[:N - W], w3a_ref[...], preferred_element_type=jnp.float32)
    t_bot = jnp.dot(x9[W:], w3c_ref[...], preferred_element_type=jnp.float32)
    zpad = jnp.zeros((W, hid), jnp.float32)
    conv = conv + jnp.concatenate([zpad, t_top], axis=0)
    conv = conv + jnp.concatenate([t_bot, zpad], axis=0)
    conv = conv + bias_ref[...]

    inv_sqrt2 = jnp.float32(0.7071067811865476)
    g = 0.5 * conv * (1.0 + jax.lax.erf(conv * inv_sqrt2))

    out = jnp.dot(g.astype(w2_ref.dtype), w2_ref[...],
                  preferred_element_type=jnp.float32)
    o_ref[0] = (out + b2_ref[...]).astype(o_ref.dtype)


def _fused_ffn(x, w3a, w3b, w3c, bias_field, w2, b2, *, H, W,
               interpret=False):
    B, N, C = x.shape
    hid = w2.shape[0]
    assert N == H * W

    def body(*refs):
        _ffn_kernel(*refs, H=H, W=W)

    return pl.pallas_call(
        body,
        out_shape=jax.ShapeDtypeStruct((B, N, C), jnp.float32),
        grid_spec=pltpu.PrefetchScalarGridSpec(
            num_scalar_prefetch=0,
            grid=(B,),
            in_specs=[
                pl.BlockSpec((1, N, C), lambda b: (b, 0, 0)),
                pl.BlockSpec((3 * C, hid), lambda b: (0, 0)),
                pl.BlockSpec((3 * C, hid), lambda b: (0, 0)),
                pl.BlockSpec((3 * C, hid), lambda b: (0, 0)),
                pl.BlockSpec((N, hid), lambda b: (0, 0)),
                pl.BlockSpec((hid, C), lambda b: (0, 0)),
                pl.BlockSpec((1, C), lambda b: (0, 0)),
            ],
            out_specs=pl.BlockSpec((1, N, C), lambda b: (b, 0, 0)),
        ),
        compiler_params=pltpu.CompilerParams(
            dimension_semantics=("parallel",),
            vmem_limit_bytes=100 * 1024 * 1024,
        ),
        cost_estimate=pl.CostEstimate(
            flops=2 * B * N * 3 * C * hid * 3 + 2 * B * N * hid * C,
            transcendentals=B * N * hid,
            bytes_accessed=(B * N * C * 2 + B * N * C * 4 + N * hid * 4
                            + (9 * C * hid + hid * C) * 2),
        ),
        interpret=interpret,
    )(x, w3a, w3b, w3c, bias_field, w2, b2.reshape(1, C).astype(jnp.float32))


def _prep_weights(w1, b1, dw_w, dw_b, H, W):
    w3 = (w1[None, None] * dw_w[:, :, None, :]).astype(jnp.bfloat16)
    C, hid = w1.shape
    w3 = w3.reshape(3, 3 * C, hid)

    ksum = dw_w.sum((0, 1))
    row0, row2 = dw_w[0].sum(0), dw_w[2].sum(0)
    col0, col2 = dw_w[:, 0].sum(0), dw_w[:, 2].sum(0)
    eh = jnp.zeros((H, 1, 1), jnp.float32)
    top = eh.at[0].set(1.0)
    bot = eh.at[H - 1].set(1.0)
    ew = jnp.zeros((1, W, 1), jnp.float32)
    lef = ew.at[:, 0].set(1.0)
    rig = ew.at[:, W - 1].set(1.0)
    miss = (top * row0 + bot * row2 + lef * col0 + rig * col2
            - top * lef * dw_w[0, 0] - top * rig * dw_w[0, 2]
            - bot * lef * dw_w[2, 0] - bot * rig * dw_w[2, 2])
    bias_field = dw_b + b1 * (ksum - miss)
    return w3[0], w3[1], w3[2], bias_field.reshape(H * W, hid)


def kernel(x, w1, b1, w2, b2, dw_w, dw_b):
    B, N, C = x.shape
    H = W = math.isqrt(N)
    w3a, w3b, w3c, bias_field = _prep_weights(w1, b1, dw_w, dw_b, H, W)
    return _fused_ffn(
        x.astype(jnp.bfloat16), w3a, w3b, w3c, bias_field,
        w2.astype(jnp.bfloat16), b2, H=H, W=W,
    )


# speedup vs baseline: 2.6793x; 1.3291x over previous
import math

import jax
import jax.numpy as jnp
from jax.experimental import pallas as pl
from jax.experimental.pallas import tpu as pltpu


def _ffn_kernel(x_ref, w27_ref, bias_ref, w2_ref, b2_ref, o_ref, *, H, W):
    N = H * W
    C = x_ref.shape[-1]

    x3 = x_ref[0].astype(jnp.bfloat16).reshape(H, W, C)
    zc = jnp.zeros((H, 1, C), x3.dtype)
    xm = jnp.concatenate([zc, x3[:, :W - 1]], axis=1)
    xp = jnp.concatenate([x3[:, 1:], zc], axis=1)
    x9 = jnp.concatenate([xm, x3, xp], axis=2)

    zr = jnp.zeros((1, W, 3 * C), x3.dtype)
    xdn = jnp.concatenate([zr, x9[:H - 1]], axis=0)
    xup = jnp.concatenate([x9[1:], zr], axis=0)
    x27 = jnp.concatenate([xdn, x9, xup], axis=2).reshape(N, 9 * C)

    conv = jnp.dot(x27, w27_ref[...], preferred_element_type=jnp.float32)
    conv = conv + bias_ref[...]

    inv_sqrt2 = jnp.float32(0.7071067811865476)
    g = 0.5 * conv * (1.0 + jax.lax.erf(conv * inv_sqrt2))

    out = jnp.dot(g.astype(w2_ref.dtype), w2_ref[...],
                  preferred_element_type=jnp.float32)
    o_ref[0] = (out + b2_ref[...]).astype(o_ref.dtype)


def _fused_ffn(x, w27, bias_field, w2, b2, *, H, W, interpret=False):
    B, N, C = x.shape
    hid = w2.shape[0]
    assert N == H * W

    def body(*refs):
        _ffn_kernel(*refs, H=H, W=W)

    return pl.pallas_call(
        body,
        out_shape=jax.ShapeDtypeStruct((B, N, C), jnp.float32),
        grid_spec=pltpu.PrefetchScalarGridSpec(
            num_scalar_prefetch=0,
            grid=(B,),
            in_specs=[
                pl.BlockSpec((1, N, C), lambda b: (b, 0, 0)),
                pl.BlockSpec((9 * C, hid), lambda b: (0, 0)),
                pl.BlockSpec((N, hid), lambda b: (0, 0)),
                pl.BlockSpec((hid, C), lambda b: (0, 0)),
                pl.BlockSpec((1, C), lambda b: (0, 0)),
            ],
            out_specs=pl.BlockSpec((1, N, C), lambda b: (b, 0, 0)),
        ),
        compiler_params=pltpu.CompilerParams(
            dimension_semantics=("parallel",),
            vmem_limit_bytes=100 * 1024 * 1024,
        ),
        cost_estimate=pl.CostEstimate(
            flops=2 * B * N * 9 * C * hid + 2 * B * N * hid * C,
            transcendentals=B * N * hid,
            bytes_accessed=(B * N * C * 4 + B * N * C * 4 + N * hid * 4
                            + (9 * C * hid + hid * C) * 2),
        ),
        interpret=interpret,
    )(x, w27, bias_field, w2, b2.reshape(1, C).astype(jnp.float32))


def _prep_weights(w1, b1, dw_w, dw_b, H, W):
    C, hid = w1.shape
    w27 = (w1[None, None] * dw_w[:, :, None, :]).astype(jnp.bfloat16)
    w27 = w27.reshape(9 * C, hid)

    ksum = dw_w.sum((0, 1))
    row0, row2 = dw_w[0].sum(0), dw_w[2].sum(0)
    col0, col2 = dw_w[:, 0].sum(0), dw_w[:, 2].sum(0)
    eh = jnp.zeros((H, 1, 1), jnp.float32)
    top = eh.at[0].set(1.0)
    bot = eh.at[H - 1].set(1.0)
    ew = jnp.zeros((1, W, 1), jnp.float32)
    lef = ew.at[:, 0].set(1.0)
    rig = ew.at[:, W - 1].set(1.0)
    miss = (top * row0 + bot * row2 + lef * col0 + rig * col2
            - top * lef * dw_w[0, 0] - top * rig * dw_w[0, 2]
            - bot * lef * dw_w[2, 0] - bot * rig * dw_w[2, 2])
    bias_field = dw_b + b1 * (ksum - miss)
    return w27, bias_field.reshape(H * W, hid)


def kernel(x, w1, b1, w2, b2, dw_w, dw_b):
    B, N, C = x.shape
    H = W = math.isqrt(N)
    w27, bias_field = _prep_weights(w1, b1, dw_w, dw_b, H, W)
    return _fused_ffn(x, w27, bias_field, w2.astype(jnp.bfloat16), b2,
                      H=H, W=W)
